# Initial kernel scaffold; baseline (speedup 1.0000x reference)
import jax
import jax.numpy as jnp
from jax.experimental import pallas as pl


def kernel(slots, corpus_encs, corpus_ids):
    raise NotImplementedError("write your pallas kernel here")


# speedup vs baseline: 33.8359x; 33.8359x over previous
import jax
import jax.numpy as jnp
from jax.experimental import pallas as pl
from jax.experimental.pallas import tpu as pltpu

_NUM_CONCEPTS = 32
_TOPK = 5


def _knn_vote_body(q_ref, c_ref, ids_ref, qn_ref, kn_ref, codes_ref, probs_ref):
    q = q_ref[0]
    c = c_ref[0]
    ids = ids_ref[0]
    qn = qn_ref[0]
    kn = kn_ref[0]

    dots = jax.lax.dot_general(
        q, c, (((1,), (1,)), ((), ())),
        preferred_element_type=jnp.float32)
    d2 = jnp.maximum((qn + kn) - 2.0 * dots, 0.0)
    dist = jnp.sqrt(d2)

    kiota = jax.lax.broadcasted_iota(jnp.int32, dist.shape, 1)
    big_i = jnp.int32(2 ** 30)
    inf = jnp.float32(jnp.inf)

    sel = []
    for r in range(_TOPK):
        m = jnp.min(dist, axis=1, keepdims=True)
        idx = jnp.min(jnp.where(dist == m, kiota, big_i), axis=1,
                      keepdims=True)
        hit = kiota == idx
        idr = jnp.min(jnp.where(hit, ids, big_i), axis=1,
                      keepdims=True)
        sel.append(idr)
        if r < _TOPK - 1:
            dist = jnp.where(hit, inf, dist)

    ciota = jax.lax.broadcasted_iota(jnp.int32, (q.shape[0], _NUM_CONCEPTS), 1)
    counts = sel[0] == ciota
    counts = counts.astype(jnp.int32)
    for r in range(1, _TOPK):
        counts = counts + (sel[r] == ciota).astype(jnp.int32)
    maxc = jnp.max(counts, axis=1, keepdims=True)
    code = jnp.min(jnp.where(counts == maxc, ciota, jnp.int32(_NUM_CONCEPTS)),
                   axis=1)
    codes_ref[0, 0, :] = code.astype(jnp.float32)
    probs_ref[0, 0, :] = maxc[:, 0].astype(jnp.float32) * (1.0 / _TOPK)


def kernel(slots, corpus_encs, corpus_ids):
    B, S, D = slots.shape
    nb, K, bs = corpus_encs.shape
    Q = B * S

    q = jnp.transpose(slots.reshape(Q, nb, bs), (1, 0, 2))
    qn = jnp.sum(q * q, axis=-1, keepdims=True)
    kn = jnp.sum(corpus_encs * corpus_encs, axis=-1)

    ids3 = corpus_ids.reshape(nb, 1, K)
    kn3 = kn.reshape(nb, 1, K)

    codes, probs = pl.pallas_call(
        _knn_vote_body,
        grid=(nb,),
        in_specs=[
            pl.BlockSpec((1, Q, bs), lambda n: (n, 0, 0)),
            pl.BlockSpec((1, K, bs), lambda n: (n, 0, 0)),
            pl.BlockSpec((1, 1, K), lambda n: (n, 0, 0)),
            pl.BlockSpec((1, Q, 1), lambda n: (n, 0, 0)),
            pl.BlockSpec((1, 1, K), lambda n: (n, 0, 0)),
        ],
        out_specs=[
            pl.BlockSpec((1, 1, Q), lambda n: (n, 0, 0)),
            pl.BlockSpec((1, 1, Q), lambda n: (n, 0, 0)),
        ],
        out_shape=[
            jax.ShapeDtypeStruct((nb, 1, Q), jnp.float32),
            jax.ShapeDtypeStruct((nb, 1, Q), jnp.float32),
        ],
        compiler_params=pltpu.CompilerParams(
            dimension_semantics=("arbitrary",),
        ),
    )(q, corpus_encs, ids3, qn, kn3)

    codes = jnp.transpose(codes.reshape(nb, Q), (1, 0)).reshape(B, S, nb)
    probs = jnp.transpose(probs.reshape(nb, Q), (1, 0)).reshape(B, S, nb)
    return codes, probs


# speedup vs baseline: 47.9535x; 1.4172x over previous
import jax
import jax.numpy as jnp
from jax.experimental import pallas as pl
from jax.experimental.pallas import tpu as pltpu

_NUM_CONCEPTS = 32
_TOPK = 5


def _knn_vote_body(q_ref, c_ref, ids_ref, qn_ref, kn_ref, codes_ref, probs_ref):
    q = q_ref[0]
    c = c_ref[0]
    ids = ids_ref[0]
    qn = qn_ref[0]
    kn = kn_ref[0]

    dots = jax.lax.dot_general(
        q, c, (((1,), (1,)), ((), ())),
        preferred_element_type=jnp.float32)
    d2 = jnp.maximum((qn + kn) - 2.0 * dots, 0.0)
    dist = jnp.sqrt(d2)

    kiota_row = jax.lax.broadcasted_iota(jnp.int32, ids.shape, 1)
    pack_row = jnp.bitwise_or(jnp.left_shift(kiota_row, 5), ids)
    big_i = jnp.int32(2 ** 30)
    inf = jnp.float32(jnp.inf)

    sel = []
    m = jnp.min(dist, axis=1, keepdims=True)
    for r in range(_TOPK):
        pm = jnp.min(jnp.where(dist == m, pack_row, big_i), axis=1,
                     keepdims=True)
        sel.append(jnp.bitwise_and(pm, jnp.int32(31)))
        if r < _TOPK - 1:
            dist = jnp.where(pack_row == pm, inf, dist)
            m = jnp.min(dist, axis=1, keepdims=True)

    ciota = jax.lax.broadcasted_iota(jnp.int32, (q.shape[0], _NUM_CONCEPTS), 1)
    counts = sel[0] == ciota
    counts = counts.astype(jnp.int32)
    for r in range(1, _TOPK):
        counts = counts + (sel[r] == ciota).astype(jnp.int32)
    maxc = jnp.max(counts, axis=1, keepdims=True)
    code = jnp.min(jnp.where(counts == maxc, ciota, jnp.int32(_NUM_CONCEPTS)),
                   axis=1)
    codes_ref[0, 0, :] = code.astype(jnp.float32)
    probs_ref[0, 0, :] = maxc[:, 0].astype(jnp.float32) * (1.0 / _TOPK)


def kernel(slots, corpus_encs, corpus_ids):
    B, S, D = slots.shape
    nb, K, bs = corpus_encs.shape
    Q = B * S

    q = jnp.transpose(slots.reshape(Q, nb, bs), (1, 0, 2))
    qn = jnp.sum(q * q, axis=-1, keepdims=True)
    kn = jnp.sum(corpus_encs * corpus_encs, axis=-1)

    ids3 = corpus_ids.reshape(nb, 1, K)
    kn3 = kn.reshape(nb, 1, K)

    codes, probs = pl.pallas_call(
        _knn_vote_body,
        grid=(nb,),
        in_specs=[
            pl.BlockSpec((1, Q, bs), lambda n: (n, 0, 0)),
            pl.BlockSpec((1, K, bs), lambda n: (n, 0, 0)),
            pl.BlockSpec((1, 1, K), lambda n: (n, 0, 0)),
            pl.BlockSpec((1, Q, 1), lambda n: (n, 0, 0)),
            pl.BlockSpec((1, 1, K), lambda n: (n, 0, 0)),
        ],
        out_specs=[
            pl.BlockSpec((1, 1, Q), lambda n: (n, 0, 0)),
            pl.BlockSpec((1, 1, Q), lambda n: (n, 0, 0)),
        ],
        out_shape=[
            jax.ShapeDtypeStruct((nb, 1, Q), jnp.float32),
            jax.ShapeDtypeStruct((nb, 1, Q), jnp.float32),
        ],
        compiler_params=pltpu.CompilerParams(
            dimension_semantics=("arbitrary",),
        ),
    )(q, corpus_encs, ids3, qn, kn3)

    codes = jnp.transpose(codes.reshape(nb, Q), (1, 0)).reshape(B, S, nb)
    probs = jnp.transpose(probs.reshape(nb, Q), (1, 0)).reshape(B, S, nb)
    return codes, probs
